# baseline (speedup 1.0000x reference)
import functools

import jax
import jax.numpy as jnp
from jax import lax
from jax.experimental import pallas as pl
from jax.experimental.pallas import tpu as pltpu
from jax.experimental.pallas import tpu_sc as plsc

_NP = 16


_PACK = 128 // _NP


def _proj_body(t_ref, w1_ref, w2p_ref, out_ref, *, scale, h):
    rt = lax.dot_general(w1_ref[...], w2p_ref[...], (((0,), (1,)), ((), ())),
                         preferred_element_type=jnp.float32) * scale
    tbl = t_ref[...]
    n = tbl.shape[0]
    tp = jnp.concatenate(
        [tbl, jnp.zeros((n, 128 - h), jnp.float32)], axis=1)
    t8 = tp.reshape(n // _PACK, _PACK * 128)
    rtp = jnp.concatenate(
        [rt, jnp.zeros((128 - h, _NP), jnp.float32)], axis=0)
    wbig = jnp.tile(rtp, (_PACK, _PACK))
    rows = lax.broadcasted_iota(jnp.int32, wbig.shape, 0) // 128
    cols = lax.broadcasted_iota(jnp.int32, wbig.shape, 1) // _NP
    wbig = jnp.where(rows == cols, wbig, 0.0)
    res = lax.dot_general(t8, wbig, (((1,), (0,)), ((), ())),
                          preferred_element_type=jnp.float32)
    rid = lax.broadcasted_iota(jnp.int32, res.shape, 0)
    cid = lax.broadcasted_iota(jnp.int32, res.shape, 1)
    res = jnp.where((rid == 0) & (pl.program_id(0) == 0) & (cid < _NP),
                    0.0, res)
    out_ref[...] = res.astype(jnp.bfloat16)


def _project_table(table, w1, w2p, scale):
    v, h = table.shape
    vp = v // _PACK
    blk = 1600
    nsteps = (vp + blk - 1) // blk
    return pl.pallas_call(
        functools.partial(_proj_body, scale=scale, h=h),
        grid=(nsteps,),
        in_specs=[
            pl.BlockSpec((blk * _PACK, h), lambda i: (i, 0)),
            pl.BlockSpec((h, h), lambda i: (0, 0)),
            pl.BlockSpec((_NP, h), lambda i: (0, 0)),
        ],
        out_specs=pl.BlockSpec((blk, 128), lambda i: (i, 0)),
        out_shape=jax.ShapeDtypeStruct((vp, 128), jnp.bfloat16),
    )(table, w1, w2p)


def _gather_sum(text_flat, tq, batch, seq):
    info = plsc.get_sparse_core_info()
    ncores, nsub = info.num_cores, info.num_subcores
    nw = ncores * nsub
    bpw = batch // nw
    nrows = tq.shape[0]
    chunks = []
    off = 0
    while off < seq:
        sz = min(128, seq - off)
        chunks.append((off, sz))
        off += sz
    stg = (-(-nrows // nsub) + 7) // 8 * 8

    mesh = plsc.VectorSubcoreMesh(core_axis_name="c", subcore_axis_name="s")

    @functools.partial(
        pl.kernel,
        mesh=mesh,
        compiler_params=pltpu.CompilerParams(use_tc_tiling_on_sc=False),
        out_type=jax.ShapeDtypeStruct((batch, 2, _NP), jnp.bfloat16),
        scratch_types=[
            pltpu.VMEM((bpw * seq,), jnp.int32),
            pltpu.VMEM((6, seq, _NP), jnp.bfloat16),
            pltpu.VMEM((bpw, 2, _NP), jnp.bfloat16),
        ] + [pltpu.SemaphoreType.DMA] * 6,
    )
    def k(text_hbm, tq_hbm, out_hbm, idx_v, buf_v, out_v, *sems):
        wid = lax.axis_index("s") * ncores + lax.axis_index("c")
        base = wid * (bpw * seq)
        pltpu.sync_copy(text_hbm.at[pl.ds(base, bpw * seq)], idx_v)

        def issue(s, b):
            for (o, sz) in chunks:
                pltpu.async_copy(
                    tq_hbm.at[idx_v.at[pl.ds(s * seq + o, sz)]],
                    buf_v.at[b, pl.ds(o, sz)],
                    sems[b])

        def wait(b):
            for (o, sz) in chunks:
                pltpu.make_async_copy(
                    tq_hbm.at[pl.ds(0, sz)],
                    buf_v.at[b, pl.ds(o, sz)],
                    sems[b]).wait()

        def accum(s, b):
            zero = jnp.zeros((2, _NP), jnp.bfloat16)

            def body(l, accs):
                r = l * 8
                return tuple(
                    accs[u] + buf_v[b, pl.ds(r + 2 * u, 2), :]
                    for u in range(4))

            accs = lax.fori_loop(0, seq // 8, body, (zero,) * 4)
            out_v[s, :, :] = (accs[0] + accs[1]) + (accs[2] + accs[3])

        for j in range(6):
            issue(j, j)

        def body(g, _):
            s0 = g * 6
            for j in range(6):
                wait(j)
                accum(s0 + j, j)

                @pl.when(s0 + j + 6 < bpw)
                def _():
                    issue(s0 + j + 6, j)

            return 0

        lax.fori_loop(0, bpw // 6, body, 0)
        for j in range(bpw - (bpw // 6) * 6):
            wait(j)
            accum((bpw // 6) * 6 + j, j)
        pltpu.sync_copy(out_v, out_hbm.at[pl.ds(wid * bpw, bpw)])

    return k(text_flat, tq)


def _finish_body(z_ref, w2p_ref, b1_ref, b2p_ref, out_ref, *, ncls):
    c = lax.dot_general(b1_ref[...], w2p_ref[...], (((1,), (1,)), ((), ())),
                        precision=lax.Precision.HIGHEST,
                        preferred_element_type=jnp.float32) + b2p_ref[...]
    zacc = z_ref[...].astype(jnp.float32)
    z = zacc[:, 0, :] + zacc[:, 1, :] + c
    zs = z[:, :ncls]
    m = jnp.max(zs, axis=1, keepdims=True)
    e = jnp.exp(zs - m)
    out_ref[...] = (zs - m) - jnp.log(jnp.sum(e, axis=1, keepdims=True))


def _finish(zacc, w2p, b1, b2p, ncls):
    batch = zacc.shape[0]
    return pl.pallas_call(
        functools.partial(_finish_body, ncls=ncls),
        in_specs=[
            pl.BlockSpec(zacc.shape, lambda: (0, 0, 0)),
            pl.BlockSpec(w2p.shape, lambda: (0, 0)),
            pl.BlockSpec((1, b1.shape[0]), lambda: (0, 0)),
            pl.BlockSpec((1, _NP), lambda: (0, 0)),
        ],
        out_specs=pl.BlockSpec((batch, ncls), lambda: (0, 0)),
        out_shape=jax.ShapeDtypeStruct((batch, ncls), jnp.float32),
    )(zacc, w2p, b1.reshape(1, -1), b2p.reshape(1, -1))


def kernel(text, text_lengths, table, W1, b1, W2, b2):
    del text_lengths
    batch, seq = text.shape
    ncls, h = W2.shape
    w2p = jnp.zeros((_NP, h), W2.dtype).at[:ncls].set(W2)
    b2p = jnp.zeros((_NP,), b2.dtype).at[:ncls].set(b2)
    tq = _project_table(table, W1, w2p, 1.0 / seq)
    tq8 = tq.reshape(-1, _NP)
    zacc = _gather_sum(text.reshape(-1), tq8, batch, seq)
    return _finish(zacc, w2p, b1, b2p, ncls)


# speedup vs baseline: 1.0035x; 1.0035x over previous
import functools

import jax
import jax.numpy as jnp
from jax import lax
from jax.experimental import pallas as pl
from jax.experimental.pallas import tpu as pltpu
from jax.experimental.pallas import tpu_sc as plsc

_NP = 16


_PACK = 128 // _NP


def _proj_body(t_ref, w1_ref, w2p_ref, out_ref, *, scale, h):
    rt = lax.dot_general(w1_ref[...], w2p_ref[...], (((0,), (1,)), ((), ())),
                         preferred_element_type=jnp.float32) * scale
    tbl = t_ref[...]
    n = tbl.shape[0]
    tp = jnp.concatenate(
        [tbl, jnp.zeros((n, 128 - h), jnp.float32)], axis=1)
    t8 = tp.reshape(n // _PACK, _PACK * 128)
    rtp = jnp.concatenate(
        [rt, jnp.zeros((128 - h, _NP), jnp.float32)], axis=0)
    wbig = jnp.tile(rtp, (_PACK, _PACK))
    rows = lax.broadcasted_iota(jnp.int32, wbig.shape, 0) // 128
    cols = lax.broadcasted_iota(jnp.int32, wbig.shape, 1) // _NP
    wbig = jnp.where(rows == cols, wbig, 0.0)
    res = lax.dot_general(t8, wbig, (((1,), (0,)), ((), ())),
                          preferred_element_type=jnp.float32)
    rid = lax.broadcasted_iota(jnp.int32, res.shape, 0)
    cid = lax.broadcasted_iota(jnp.int32, res.shape, 1)
    res = jnp.where((rid == 0) & (pl.program_id(0) == 0) & (cid < _NP),
                    0.0, res)
    out_ref[...] = res


def _project_table(table, w1, w2p, scale):
    v, h = table.shape
    vp = v // _PACK
    blk = 1600
    nsteps = (vp + blk - 1) // blk
    return pl.pallas_call(
        functools.partial(_proj_body, scale=scale, h=h),
        grid=(nsteps,),
        in_specs=[
            pl.BlockSpec((blk * _PACK, h), lambda i: (i, 0)),
            pl.BlockSpec((h, h), lambda i: (0, 0)),
            pl.BlockSpec((_NP, h), lambda i: (0, 0)),
        ],
        out_specs=pl.BlockSpec((blk, 128), lambda i: (i, 0)),
        out_shape=jax.ShapeDtypeStruct((vp, 128), jnp.float32),
    )(table, w1, w2p)


def _gather_sum(text_flat, tq, batch, seq):
    info = plsc.get_sparse_core_info()
    ncores, nsub = info.num_cores, info.num_subcores
    nw = ncores * nsub
    bpw = batch // nw
    nrows = tq.shape[0]
    chunks = []
    off = 0
    while off < seq:
        sz = min(128, seq - off)
        chunks.append((off, sz))
        off += sz
    stg = (-(-nrows // nsub) + 7) // 8 * 8

    mesh = plsc.VectorSubcoreMesh(core_axis_name="c", subcore_axis_name="s")

    @functools.partial(
        pl.kernel,
        mesh=mesh,
        compiler_params=pltpu.CompilerParams(use_tc_tiling_on_sc=False),
        out_type=jax.ShapeDtypeStruct((batch, 2, _NP), jnp.bfloat16),
        scratch_types=[
            pltpu.VMEM((bpw * seq,), jnp.int32),
            pltpu.VMEM((6, seq, _NP), jnp.bfloat16),
            pltpu.VMEM((bpw, 2, _NP), jnp.bfloat16),
        ] + [pltpu.SemaphoreType.DMA] * 6,
    )
    def k(text_hbm, tq_hbm, out_hbm, idx_v, buf_v, out_v, *sems):
        wid = lax.axis_index("s") * ncores + lax.axis_index("c")
        base = wid * (bpw * seq)
        pltpu.sync_copy(text_hbm.at[pl.ds(base, bpw * seq)], idx_v)

        def issue(s, b):
            for (o, sz) in chunks:
                pltpu.async_copy(
                    tq_hbm.at[idx_v.at[pl.ds(s * seq + o, sz)]],
                    buf_v.at[b, pl.ds(o, sz)],
                    sems[b])

        def wait(b):
            for (o, sz) in chunks:
                pltpu.make_async_copy(
                    tq_hbm.at[pl.ds(0, sz)],
                    buf_v.at[b, pl.ds(o, sz)],
                    sems[b]).wait()

        def accum(s, b):
            zero = jnp.zeros((2, _NP), jnp.bfloat16)

            def body(l, accs):
                r = l * 8
                return tuple(
                    accs[u] + buf_v[b, pl.ds(r + 2 * u, 2), :]
                    for u in range(4))

            accs = lax.fori_loop(0, seq // 8, body, (zero,) * 4)
            out_v[s, :, :] = (accs[0] + accs[1]) + (accs[2] + accs[3])

        for j in range(6):
            issue(j, j)

        def body(g, _):
            s0 = g * 6
            for j in range(6):
                wait(j)
                accum(s0 + j, j)

                @pl.when(s0 + j + 6 < bpw)
                def _():
                    issue(s0 + j + 6, j)

            return 0

        lax.fori_loop(0, bpw // 6, body, 0)
        for j in range(bpw - (bpw // 6) * 6):
            wait(j)
            accum((bpw // 6) * 6 + j, j)
        pltpu.sync_copy(out_v, out_hbm.at[pl.ds(wid * bpw, bpw)])

    return k(text_flat, tq)


def _finish_body(z_ref, w2p_ref, b1_ref, b2p_ref, out_ref, *, ncls):
    c = lax.dot_general(b1_ref[...], w2p_ref[...], (((1,), (1,)), ((), ())),
                        precision=lax.Precision.HIGHEST,
                        preferred_element_type=jnp.float32) + b2p_ref[...]
    zacc = z_ref[...].astype(jnp.float32)
    z = zacc[:, 0, :] + zacc[:, 1, :] + c
    zs = z[:, :ncls]
    m = jnp.max(zs, axis=1, keepdims=True)
    e = jnp.exp(zs - m)
    out_ref[...] = (zs - m) - jnp.log(jnp.sum(e, axis=1, keepdims=True))


def _finish(zacc, w2p, b1, b2p, ncls):
    batch = zacc.shape[0]
    return pl.pallas_call(
        functools.partial(_finish_body, ncls=ncls),
        in_specs=[
            pl.BlockSpec(zacc.shape, lambda: (0, 0, 0)),
            pl.BlockSpec(w2p.shape, lambda: (0, 0)),
            pl.BlockSpec((1, b1.shape[0]), lambda: (0, 0)),
            pl.BlockSpec((1, _NP), lambda: (0, 0)),
        ],
        out_specs=pl.BlockSpec((batch, ncls), lambda: (0, 0)),
        out_shape=jax.ShapeDtypeStruct((batch, ncls), jnp.float32),
    )(zacc, w2p, b1.reshape(1, -1), b2p.reshape(1, -1))


def kernel(text, text_lengths, table, W1, b1, W2, b2):
    del text_lengths
    batch, seq = text.shape
    ncls, h = W2.shape
    w2p = jnp.zeros((_NP, h), W2.dtype).at[:ncls].set(W2)
    b2p = jnp.zeros((_NP,), b2.dtype).at[:ncls].set(b2)
    tq = _project_table(table, W1, w2p, 1.0 / seq)
    tq8 = tq.astype(jnp.bfloat16).reshape(-1, _NP)
    zacc = _gather_sum(text.reshape(-1), tq8, batch, seq)
    return _finish(zacc, w2p, b1, b2p, ncls)


# speedup vs baseline: 1.2178x; 1.2136x over previous
import functools

import jax
import jax.numpy as jnp
from jax import lax
from jax.experimental import pallas as pl
from jax.experimental.pallas import tpu as pltpu
from jax.experimental.pallas import tpu_sc as plsc

_NP = 16


_PACK = 128 // _NP


def _proj_body(t_ref, w1_ref, w2p_ref, out_ref, *, scale, h):
    rt = lax.dot_general(w1_ref[...], w2p_ref[...], (((0,), (1,)), ((), ())),
                         preferred_element_type=jnp.float32) * scale
    tbl = t_ref[...]
    n = tbl.shape[0]
    tp = jnp.concatenate(
        [tbl, jnp.zeros((n, 128 - h), jnp.float32)], axis=1)
    t8 = tp.reshape(n // _PACK, _PACK * 128)
    rtp = jnp.concatenate(
        [rt, jnp.zeros((128 - h, _NP), jnp.float32)], axis=0)
    wbig = jnp.tile(rtp, (_PACK, _PACK))
    rows = lax.broadcasted_iota(jnp.int32, wbig.shape, 0) // 128
    cols = lax.broadcasted_iota(jnp.int32, wbig.shape, 1) // _NP
    wbig = jnp.where(rows == cols, wbig, 0.0)
    res = lax.dot_general(t8, wbig, (((1,), (0,)), ((), ())),
                          preferred_element_type=jnp.float32)
    rid = lax.broadcasted_iota(jnp.int32, res.shape, 0)
    cid = lax.broadcasted_iota(jnp.int32, res.shape, 1)
    res = jnp.where((rid == 0) & (pl.program_id(0) == 0) & (cid < _NP),
                    0.0, res)
    out_ref[...] = res


def _project_table(table, w1, w2p, scale):
    v, h = table.shape
    vp = v // _PACK
    blk = 1600
    nsteps = (vp + blk - 1) // blk
    return pl.pallas_call(
        functools.partial(_proj_body, scale=scale, h=h),
        grid=(nsteps,),
        in_specs=[
            pl.BlockSpec((blk * _PACK, h), lambda i: (i, 0)),
            pl.BlockSpec((h, h), lambda i: (0, 0)),
            pl.BlockSpec((_NP, h), lambda i: (0, 0)),
        ],
        out_specs=pl.BlockSpec((blk, 128), lambda i: (i, 0)),
        out_shape=jax.ShapeDtypeStruct((vp, 128), jnp.float32),
    )(table, w1, w2p)


def _gather_sum(text_flat, tq, batch, seq):
    info = plsc.get_sparse_core_info()
    ncores, nsub = info.num_cores, info.num_subcores
    nw = ncores * nsub
    bpw = batch // nw
    nrows = tq.shape[0]
    chunks = []
    off = 0
    while off < seq:
        sz = min(128, seq - off)
        chunks.append((off, sz))
        off += sz
    stg = (-(-nrows // nsub) + 7) // 8 * 8

    mesh = plsc.VectorSubcoreMesh(core_axis_name="c", subcore_axis_name="s")

    @functools.partial(
        pl.kernel,
        mesh=mesh,
        compiler_params=pltpu.CompilerParams(use_tc_tiling_on_sc=False),
        out_type=jax.ShapeDtypeStruct((batch, _NP), jnp.float32),
        scratch_types=[
            pltpu.VMEM((bpw * seq,), jnp.int32),
            pltpu.VMEM((6, seq, _NP), jnp.float32),
            pltpu.VMEM((bpw, _NP), jnp.float32),
        ] + [pltpu.SemaphoreType.DMA] * 6,
    )
    def k(text_hbm, tq_hbm, out_hbm, idx_v, buf_v, out_v, *sems):
        wid = lax.axis_index("s") * ncores + lax.axis_index("c")
        base = wid * (bpw * seq)
        pltpu.sync_copy(text_hbm.at[pl.ds(base, bpw * seq)], idx_v)

        def issue(s, b):
            for (o, sz) in chunks:
                pltpu.async_copy(
                    tq_hbm.at[idx_v.at[pl.ds(s * seq + o, sz)]],
                    buf_v.at[b, pl.ds(o, sz)],
                    sems[b])

        def wait(b):
            for (o, sz) in chunks:
                pltpu.make_async_copy(
                    tq_hbm.at[pl.ds(0, sz)],
                    buf_v.at[b, pl.ds(o, sz)],
                    sems[b]).wait()

        def accum(s, b):
            zero = jnp.zeros((_NP,), jnp.float32)

            def body(l, accs):
                r = l * 8
                return tuple(accs[u] + buf_v[b, r + u, :] for u in range(8))

            accs = lax.fori_loop(0, seq // 8, body, (zero,) * 8)
            out_v[s, :] = sum(accs[1:], accs[0])

        for j in range(6):
            issue(j, j)

        def body(g, _):
            s0 = g * 6
            for j in range(6):
                wait(j)
                accum(s0 + j, j)

                @pl.when(s0 + j + 6 < bpw)
                def _():
                    issue(s0 + j + 6, j)

            return 0

        lax.fori_loop(0, bpw // 6, body, 0)
        for j in range(bpw - (bpw // 6) * 6):
            wait(j)
            accum((bpw // 6) * 6 + j, j)
        pltpu.sync_copy(out_v, out_hbm.at[pl.ds(wid * bpw, bpw)])

    return k(text_flat, tq)


def _finish_body(z_ref, w2p_ref, b1_ref, b2p_ref, out_ref, *, ncls):
    c = lax.dot_general(b1_ref[...], w2p_ref[...], (((1,), (1,)), ((), ())),
                        precision=lax.Precision.HIGHEST,
                        preferred_element_type=jnp.float32) + b2p_ref[...]
    z = z_ref[...] + c
    zs = z[:, :ncls]
    m = jnp.max(zs, axis=1, keepdims=True)
    e = jnp.exp(zs - m)
    out_ref[...] = (zs - m) - jnp.log(jnp.sum(e, axis=1, keepdims=True))


def _finish(zacc, w2p, b1, b2p, ncls):
    batch = zacc.shape[0]
    return pl.pallas_call(
        functools.partial(_finish_body, ncls=ncls),
        in_specs=[
            pl.BlockSpec(zacc.shape, lambda: (0, 0)),
            pl.BlockSpec(w2p.shape, lambda: (0, 0)),
            pl.BlockSpec((1, b1.shape[0]), lambda: (0, 0)),
            pl.BlockSpec((1, _NP), lambda: (0, 0)),
        ],
        out_specs=pl.BlockSpec((batch, ncls), lambda: (0, 0)),
        out_shape=jax.ShapeDtypeStruct((batch, ncls), jnp.float32),
    )(zacc, w2p, b1.reshape(1, -1), b2p.reshape(1, -1))


def kernel(text, text_lengths, table, W1, b1, W2, b2):
    del text_lengths
    batch, seq = text.shape
    ncls, h = W2.shape
    w2p = jnp.zeros((_NP, h), W2.dtype).at[:ncls].set(W2)
    b2p = jnp.zeros((_NP,), b2.dtype).at[:ncls].set(b2)
    tq = _project_table(table, W1, w2p, 1.0 / seq)
    tq8 = tq.reshape(-1, _NP)
    zacc = _gather_sum(text.reshape(-1), tq8, batch, seq)
    return _finish(zacc, w2p, b1, b2p, ncls)
